# baseline (speedup 1.0000x reference)
import functools

import jax
import jax.numpy as jnp
from jax import lax
from jax.experimental import pallas as pl
from jax.experimental.pallas import tpu as pltpu
from jax.experimental.pallas import tpu_sc as plsc

NC = 2
NS = 16
NW = NC * NS
LANES = 16
BC = 128
WIN = 8
KIDX = 2
NB = 2
NBG = 4
BLK = 16


def _make_kernel(B, S, D, V):
    n_win = S // WIN
    blocks = S // BLK
    TD = D // 8
    TB = B // BC

    mesh = plsc.VectorSubcoreMesh(core_axis_name="c", subcore_axis_name="s")

    @functools.partial(
        pl.kernel,
        out_type=jax.ShapeDtypeStruct((S, TD, TB, 8, BC), jnp.float32),
        mesh=mesh,
        compiler_params=pltpu.CompilerParams(
            use_tc_tiling_on_sc=False, needs_layout_passes=False),
        scratch_types=[
            pltpu.VMEM((S, D), jnp.float32),
            pltpu.VMEM((KIDX, WIN, BC), jnp.int32),
            pltpu.VMEM((NBG, BC, D), jnp.float32),
            pltpu.VMEM((NB, D, BC), jnp.float32),
            pltpu.SemaphoreType.DMA((KIDX,)),
            pltpu.SemaphoreType.DMA((NBG,)),
            pltpu.SemaphoreType.DMA((NB,)),
        ],
    )
    def emb_kernel(idx_hbm, pos_hbm, table_hbm, out_hbm,
                   pos_v, idx_v, rows_v, trans_v, si, sg, so):
        cid = lax.axis_index("c")
        sid = lax.axis_index("s")
        wid = sid * NC + cid
        b0 = wid * BC

        pltpu.sync_copy(pos_hbm, pos_v)

        def idx_copy(w, k):
            return pltpu.make_async_copy(
                idx_hbm.at[pl.ds(w * WIN, WIN), pl.ds(b0, BC)],
                idx_v.at[k], si.at[k])

        def gather_copy(l, crel):
            return pltpu.make_async_copy(
                table_hbm.at[idx_v.at[(crel // WIN) % KIDX, crel % WIN]],
                rows_v.at[crel % NBG], sg.at[crel % NBG])

        def out_descs(l, crel):
            bb = crel % NB
            return [pltpu.make_async_copy(
                        trans_v.at[bb, pl.ds(td * 8, 8)],
                        out_hbm.at[l, td, wid],
                        so.at[bb])
                    for td in range(TD)]

        def out_start(l, crel):
            for d in out_descs(l, crel):
                d.start()

        def out_wait(l, crel):
            for d in out_descs(l, crel):
                d.wait()

        iota = lax.iota(jnp.int32, LANES)
        dvecs = [d0 + iota for d0 in range(0, D, LANES)]

        def transpose_add(l, crel):
            bb = crel % NB
            bg = crel % NBG
            pregs = [pos_v[l, pl.ds(d0, LANES)] for d0 in range(0, D, LANES)]

            @plsc.parallel_loop(0, BC, step=2, unroll=2)
            def _(r):
                for rr in (0, 1):
                    col = jnp.full((LANES,), 0, jnp.int32) + (r + rr)
                    for i, d0 in enumerate(range(0, D, LANES)):
                        val = rows_v[bg, r + rr, pl.ds(d0, LANES)] + pregs[i]
                        plsc.store_scatter(trans_v.at[bb],
                                           [dvecs[i], col], val)

        def chunk(blk, crel, tail):
            l = blk * BLK + crel
            gather_copy(l, crel).wait()

            if tail:
                out_wait(l - NB, (crel - NB) % BLK)
            else:
                @pl.when(l >= NB)
                def _():
                    out_wait(l - NB, (crel - NB) % BLK)

            transpose_add(l, crel)
            out_start(l, crel)

            if tail:
                if l + NBG < S:
                    gather_copy(l + NBG, (crel + NBG) % BLK).start()
            else:
                @pl.when(l + NBG < S)
                def _():
                    if (crel + NBG) % WIN == 0:
                        idx_copy((blk * BLK + crel + NBG) // WIN,
                                 ((crel + NBG) // WIN) % KIDX).wait()
                    gather_copy(l + NBG, (crel + NBG) % BLK).start()

            if not tail and crel % WIN == WIN - 1:
                w = blk * (BLK // WIN) + crel // WIN

                @pl.when(w + KIDX < n_win)
                def _():
                    idx_copy(w + KIDX, crel // WIN).start()

        for k in range(KIDX):
            idx_copy(k, k).start()
        idx_copy(0, 0).wait()
        for crel in range(NBG):
            gather_copy(crel, crel).start()

        def block_body(blk, carry):
            for crel in range(BLK):
                chunk(blk, crel, tail=False)
            return carry
        lax.fori_loop(0, blocks, block_body, 0, unroll=False)

        for crel in range(S - blocks * BLK):
            chunk(blocks, crel, tail=True)

        for crel in range(NB):
            out_wait(S - NB + crel, (S - blocks * BLK - NB + crel))

    return emb_kernel


def kernel(x, embedding, pos_encoding):
    B, S = x.shape
    V, D = embedding.shape
    xT = jnp.swapaxes(x, 0, 1).astype(jnp.int32)
    out5 = _make_kernel(B, S, D, V)(xT, pos_encoding, embedding)
    return jnp.transpose(out5, (2, 4, 0, 1, 3)).reshape(B, S, D)


# speedup vs baseline: 1.0068x; 1.0068x over previous
import functools

import jax
import jax.numpy as jnp
from jax import lax
from jax.experimental import pallas as pl
from jax.experimental.pallas import tpu as pltpu
from jax.experimental.pallas import tpu_sc as plsc

NC = 2
NS = 16
NW = NC * NS
LANES = 16
BC = 128
WIN = 8
KIDX = 2
NB = 2
NBG = 4
BLK = 16


def _make_kernel(B, S, D, V):
    n_win = S // WIN
    blocks = S // BLK
    TD = D // 8
    TB = B // BC

    mesh = plsc.VectorSubcoreMesh(core_axis_name="c", subcore_axis_name="s")

    @functools.partial(
        pl.kernel,
        out_type=jax.ShapeDtypeStruct((S, TD, TB, 8, BC), jnp.float32),
        mesh=mesh,
        compiler_params=pltpu.CompilerParams(
            use_tc_tiling_on_sc=False, needs_layout_passes=False),
        scratch_types=[
            pltpu.VMEM((S, D), jnp.float32),
            pltpu.VMEM((KIDX, WIN, BC), jnp.int32),
            pltpu.VMEM((NBG, BC, D), jnp.float32),
            pltpu.VMEM((NB, D, BC), jnp.float32),
            pltpu.SemaphoreType.DMA((KIDX,)),
            pltpu.SemaphoreType.DMA((NBG,)),
            pltpu.SemaphoreType.DMA((NB,)),
        ],
    )
    def emb_kernel(idx_hbm, pos_hbm, table_hbm, out_hbm,
                   pos_v, idx_v, rows_v, trans_v, si, sg, so):
        cid = lax.axis_index("c")
        sid = lax.axis_index("s")
        wid = sid * NC + cid
        b0 = wid * BC

        pltpu.sync_copy(pos_hbm, pos_v)

        def idx_copy(w, k):
            return pltpu.make_async_copy(
                idx_hbm.at[pl.ds(w * WIN, WIN), pl.ds(b0, BC)],
                idx_v.at[k], si.at[k])

        def gather_copy(l, crel):
            return pltpu.make_async_copy(
                table_hbm.at[idx_v.at[(crel // WIN) % KIDX, crel % WIN]],
                rows_v.at[crel % NBG], sg.at[crel % NBG])

        def out_descs(l, crel):
            bb = crel % NB
            return [pltpu.make_async_copy(
                        trans_v.at[bb, pl.ds(td * 8, 8)],
                        out_hbm.at[l, td, wid],
                        so.at[bb])
                    for td in range(TD)]

        def out_start(l, crel):
            for d in out_descs(l, crel):
                d.start()

        def out_wait(l, crel):
            for d in out_descs(l, crel):
                d.wait()

        iota = lax.iota(jnp.int32, LANES)
        dvecs = [d0 + iota for d0 in range(0, D, LANES)]

        def transpose_add(l, crel):
            bb = crel % NB
            bg = crel % NBG
            pregs = [pos_v[l, pl.ds(d0, LANES)] for d0 in range(0, D, LANES)]

            @plsc.parallel_loop(0, BC, step=2, unroll=4,
                                carry=jnp.zeros((LANES,), jnp.int32))
            def _(r, col):
                for rr in (0, 1):
                    for i, d0 in enumerate(range(0, D, LANES)):
                        val = rows_v[bg, r + rr, pl.ds(d0, LANES)] + pregs[i]
                        plsc.store_scatter(trans_v.at[bb],
                                           [dvecs[i], col + rr], val)
                return col + 2

        def chunk(blk, crel, tail):
            l = blk * BLK + crel
            gather_copy(l, crel).wait()

            if tail:
                out_wait(l - NB, (crel - NB) % BLK)
            else:
                @pl.when(l >= NB)
                def _():
                    out_wait(l - NB, (crel - NB) % BLK)

            transpose_add(l, crel)
            out_start(l, crel)

            if tail:
                if l + NBG < S:
                    gather_copy(l + NBG, (crel + NBG) % BLK).start()
            else:
                @pl.when(l + NBG < S)
                def _():
                    if (crel + NBG) % WIN == 0:
                        idx_copy((blk * BLK + crel + NBG) // WIN,
                                 ((crel + NBG) // WIN) % KIDX).wait()
                    gather_copy(l + NBG, (crel + NBG) % BLK).start()

            if not tail and crel % WIN == WIN - 1:
                w = blk * (BLK // WIN) + crel // WIN

                @pl.when(w + KIDX < n_win)
                def _():
                    idx_copy(w + KIDX, crel // WIN).start()

        for k in range(KIDX):
            idx_copy(k, k).start()
        idx_copy(0, 0).wait()
        for crel in range(NBG):
            gather_copy(crel, crel).start()

        def block_body(blk, carry):
            for crel in range(BLK):
                chunk(blk, crel, tail=False)
            return carry
        lax.fori_loop(0, blocks, block_body, 0, unroll=False)

        for crel in range(S - blocks * BLK):
            chunk(blocks, crel, tail=True)

        for crel in range(NB):
            out_wait(S - NB + crel, (S - blocks * BLK - NB + crel))

    return emb_kernel


def kernel(x, embedding, pos_encoding):
    B, S = x.shape
    V, D = embedding.shape
    xT = jnp.swapaxes(x, 0, 1).astype(jnp.int32)
    out5 = _make_kernel(B, S, D, V)(xT, pos_encoding, embedding)
    return jnp.transpose(out5, (2, 4, 0, 1, 3)).reshape(B, S, D)


# speedup vs baseline: 3.5011x; 3.4774x over previous
import functools

import jax
import jax.numpy as jnp
from jax import lax
from jax.experimental import pallas as pl
from jax.experimental.pallas import tpu as pltpu
from jax.experimental.pallas import tpu_sc as plsc

NC = 2
NS = 16
NW = NC * NS
LANES = 16
BC = 128
WIN = 8
KIDX = 2
NB = 2
NBG = 4
BLK = 16


def _make_kernel(B, S, D, V):
    n_win = S // WIN
    blocks = S // BLK
    TD = D // 8
    TB = B // BC

    mesh = plsc.VectorSubcoreMesh(core_axis_name="c", subcore_axis_name="s")

    @functools.partial(
        pl.kernel,
        out_type=jax.ShapeDtypeStruct((S, TD, TB, 8, BC), jnp.float32),
        mesh=mesh,
        compiler_params=pltpu.CompilerParams(
            use_tc_tiling_on_sc=False, needs_layout_passes=False),
        scratch_types=[
            pltpu.VMEM((S, D), jnp.float32),
            pltpu.VMEM((KIDX, WIN, BC), jnp.int32),
            pltpu.VMEM((NBG, BC, D), jnp.float32),
            pltpu.VMEM((NB, D, BC), jnp.float32),
            pltpu.SemaphoreType.DMA((KIDX,)),
            pltpu.SemaphoreType.DMA((NBG,)),
            pltpu.SemaphoreType.DMA((NB,)),
        ],
    )
    def emb_kernel(idx_hbm, pos_hbm, table_hbm, out_hbm,
                   pos_v, idx_v, rows_v, trans_v, si, sg, so):
        cid = lax.axis_index("c")
        sid = lax.axis_index("s")
        wid = sid * NC + cid
        b0 = wid * BC

        pltpu.sync_copy(pos_hbm, pos_v)

        def idx_copy(w, k):
            return pltpu.make_async_copy(
                idx_hbm.at[pl.ds(w * WIN, WIN), pl.ds(b0, BC)],
                idx_v.at[k], si.at[k])

        def gather_copy(l, crel):
            return pltpu.make_async_copy(
                table_hbm.at[idx_v.at[(crel // WIN) % KIDX, crel % WIN]],
                rows_v.at[crel % NBG], sg.at[crel % NBG])

        def out_descs(l, crel):
            bb = crel % NB
            return [pltpu.make_async_copy(
                        trans_v.at[bb, pl.ds(td * 8, 8)],
                        out_hbm.at[l, td, wid],
                        so.at[bb])
                    for td in range(TD)]

        def out_start(l, crel):
            for d in out_descs(l, crel):
                d.start()

        def out_wait(l, crel):
            for d in out_descs(l, crel):
                d.wait()

        iota = lax.iota(jnp.int32, LANES)
        dvecs = [d0 + iota for d0 in range(0, D, LANES)]

        def transpose_add(l, crel):
            bb = crel % NB
            bg = crel % NBG
            pregs = [pos_v[l, pl.ds(d0, LANES)] for d0 in range(0, D, LANES)]

            @plsc.parallel_loop(0, BC, step=1, unroll=2)
            def _(m):
                tokvec = (m & ~(LANES - 1)) + ((iota + (m & (LANES - 1)))
                                               & (LANES - 1))
                for i, d0 in enumerate(range(0, D, LANES)):
                    val = plsc.load_gather(rows_v.at[bg], [tokvec, dvecs[i]])
                    plsc.store_scatter(trans_v.at[bb], [dvecs[i], tokvec],
                                       val + pregs[i])

        def chunk(blk, crel, tail):
            l = blk * BLK + crel
            gather_copy(l, crel).wait()

            if tail:
                out_wait(l - NB, (crel - NB) % BLK)
            else:
                @pl.when(l >= NB)
                def _():
                    out_wait(l - NB, (crel - NB) % BLK)

            transpose_add(l, crel)
            out_start(l, crel)

            if tail:
                if l + NBG < S:
                    gather_copy(l + NBG, (crel + NBG) % BLK).start()
            else:
                @pl.when(l + NBG < S)
                def _():
                    if (crel + NBG) % WIN == 0:
                        idx_copy((blk * BLK + crel + NBG) // WIN,
                                 ((crel + NBG) // WIN) % KIDX).wait()
                    gather_copy(l + NBG, (crel + NBG) % BLK).start()

            if not tail and crel % WIN == WIN - 1:
                w = blk * (BLK // WIN) + crel // WIN

                @pl.when(w + KIDX < n_win)
                def _():
                    idx_copy(w + KIDX, crel // WIN).start()

        for k in range(KIDX):
            idx_copy(k, k).start()
        idx_copy(0, 0).wait()
        for crel in range(NBG):
            gather_copy(crel, crel).start()

        def block_body(blk, carry):
            for crel in range(BLK):
                chunk(blk, crel, tail=False)
            return carry
        lax.fori_loop(0, blocks, block_body, 0, unroll=False)

        for crel in range(S - blocks * BLK):
            chunk(blocks, crel, tail=True)

        for crel in range(NB):
            out_wait(S - NB + crel, (S - blocks * BLK - NB + crel))

    return emb_kernel


def kernel(x, embedding, pos_encoding):
    B, S = x.shape
    V, D = embedding.shape
    xT = jnp.swapaxes(x, 0, 1).astype(jnp.int32)
    out5 = _make_kernel(B, S, D, V)(xT, pos_encoding, embedding)
    return jnp.transpose(out5, (2, 4, 0, 1, 3)).reshape(B, S, D)
